# Initial kernel scaffold; baseline (speedup 1.0000x reference)
import jax
import jax.numpy as jnp
from jax.experimental import pallas as pl


def kernel(x, edge_index, batch, W1_rel, W1_root, b1, W2_rel, W2_root, b2, W_fc, b_fc):
    raise NotImplementedError("write your pallas kernel here")


# speedup vs baseline: 10.9400x; 10.9400x over previous
import functools

import jax
import jax.numpy as jnp
from jax import lax
from jax.experimental import pallas as pl
from jax.experimental.pallas import tpu as pltpu
from jax.experimental.pallas import tpu_sc as plsc

N_CORES = 2
N_SUB = 16
NW = N_CORES * N_SUB
CH = 128


def _mm_split_kern(x_ref, w_ref, y_ref, r_ref, *, split):
    prod = jnp.dot(x_ref[...], w_ref[...], preferred_element_type=jnp.float32)
    y_ref[...] = prod[:, :split]
    r_ref[...] = prod[:, split:]


def _combine_mm_kern(p0_ref, p1_ref, r_ref, b_ref, w_ref, y_ref, r2_ref, *, split):
    h = jnp.maximum(p0_ref[0] + p1_ref[0] + r_ref[...] + b_ref[...], 0.0)
    prod = jnp.dot(h, w_ref[...], preferred_element_type=jnp.float32)
    y_ref[...] = prod[:, :split]
    r2_ref[...] = prod[:, split:]


def _final_kern(p0_ref, p1_ref, r_ref, b_ref, batch_ref, wfc_ref, bfc_ref,
                out_ref, *, n_graphs):
    h = jnp.maximum(p0_ref[0] + p1_ref[0] + r_ref[...] + b_ref[...], 0.0)
    n = h.shape[0]
    gids = lax.broadcasted_iota(jnp.int32, (n_graphs, n), 0)
    onehot = (batch_ref[...] == gids).astype(jnp.float32)
    sums = jnp.dot(onehot, h, preferred_element_type=jnp.float32)
    counts = jnp.sum(onehot, axis=1, keepdims=True)
    pooled = sums / jnp.maximum(counts, 1.0)
    logits = jnp.dot(pooled, wfc_ref[...], preferred_element_type=jnp.float32)
    logits = logits + bfc_ref[...]
    m = jnp.max(logits, axis=1, keepdims=True)
    lse = m + jnp.log(jnp.sum(jnp.exp(logits - m), axis=1, keepdims=True))
    out_ref[...] = logits - lse


def _acc_rows(n_nodes):
    return -(-(n_nodes + 1) // 128) * 128


def _make_edge_aggregate(n_nodes, feat, ec):
    a_rows = _acc_rows(n_nodes)
    zrows = a_rows // N_SUB
    mesh = plsc.VectorSubcoreMesh(core_axis_name="c", subcore_axis_name="s")

    @functools.partial(
        pl.kernel,
        mesh=mesh,
        compiler_params=pltpu.CompilerParams(use_tc_tiling_on_sc=False),
        out_type=jax.ShapeDtypeStruct((N_CORES, a_rows, feat), jnp.float32),
        scratch_types=[
            pltpu.VMEM((ec, CH), jnp.int32),
            pltpu.VMEM((ec, CH), jnp.int32),
            pltpu.VMEM((CH, feat), jnp.float32),
            pltpu.VMEM_SHARED((a_rows, feat), jnp.float32),
            pltpu.SemaphoreType.DMA,
        ],
    )
    def k(y_hbm, src_hbm, dst_hbm, zero_hbm, out_hbm,
          src_v, dst_v, rows_v, aggr_sh, sem):
        c = lax.axis_index("c")
        s = lax.axis_index("s")
        w = c * N_SUB + s
        pltpu.sync_copy(zero_hbm.at[pl.ds(s * zrows, zrows)],
                        aggr_sh.at[pl.ds(s * zrows, zrows)])
        pltpu.sync_copy(src_hbm.at[w], src_v)
        pltpu.sync_copy(dst_hbm.at[w], dst_v)
        plsc.subcore_barrier()

        def body(j, carry):
            pltpu.async_copy(y_hbm.at[src_v.at[j]], rows_v, sem).wait()
            pltpu.sync_copy(rows_v, aggr_sh.at[dst_v.at[j]], add=True)
            return carry

        lax.fori_loop(0, ec, body, 0)
        plsc.subcore_barrier()
        pltpu.sync_copy(aggr_sh.at[pl.ds(s * zrows, zrows)],
                        out_hbm.at[c, pl.ds(s * zrows, zrows)])

    return k


def kernel(x, edge_index, batch, W1_rel, W1_root, b1, W2_rel, W2_root, b2,
           W_fc, b_fc):
    n_nodes, d_feat = x.shape
    n_edges = edge_index.shape[1]
    hid1 = W1_rel.shape[1]
    hid2 = W2_rel.shape[1]
    n_classes = W_fc.shape[1]
    n_graphs = 64

    ec = -(-n_edges // (NW * CH))
    e_pad = NW * ec * CH
    src = edge_index[0].astype(jnp.int32)
    dst = edge_index[1].astype(jnp.int32)
    src_p = jnp.concatenate(
        [src, jnp.zeros((e_pad - n_edges,), jnp.int32)]).reshape(NW, ec, CH)
    dst_p = jnp.concatenate(
        [dst, jnp.full((e_pad - n_edges,), n_nodes, jnp.int32)]).reshape(NW, ec, CH)

    blk = 2000
    grid = n_nodes // blk

    w1 = jnp.concatenate([W1_rel, W1_root], axis=1)
    y1, r1 = pl.pallas_call(
        functools.partial(_mm_split_kern, split=hid1),
        grid=(grid,),
        in_specs=[
            pl.BlockSpec((blk, d_feat), lambda i: (i, 0)),
            pl.BlockSpec((d_feat, 2 * hid1), lambda i: (0, 0)),
        ],
        out_specs=[
            pl.BlockSpec((blk, hid1), lambda i: (i, 0)),
            pl.BlockSpec((blk, hid1), lambda i: (i, 0)),
        ],
        out_shape=[jax.ShapeDtypeStruct((n_nodes, hid1), jnp.float32)] * 2,
    )(x, w1)

    a_rows = _acc_rows(n_nodes)

    zeros1 = jnp.zeros((a_rows, hid1), jnp.float32)
    p1 = _make_edge_aggregate(n_nodes, hid1, ec)(y1, src_p, dst_p, zeros1)

    w2 = jnp.concatenate([W2_rel, W2_root], axis=1)
    y2, r2 = pl.pallas_call(
        functools.partial(_combine_mm_kern, split=hid2),
        grid=(grid,),
        in_specs=[
            pl.BlockSpec((1, blk, hid1), lambda i: (0, i, 0)),
            pl.BlockSpec((1, blk, hid1), lambda i: (1, i, 0)),
            pl.BlockSpec((blk, hid1), lambda i: (i, 0)),
            pl.BlockSpec((1, hid1), lambda i: (0, 0)),
            pl.BlockSpec((hid1, 2 * hid2), lambda i: (0, 0)),
        ],
        out_specs=[
            pl.BlockSpec((blk, hid2), lambda i: (i, 0)),
            pl.BlockSpec((blk, hid2), lambda i: (i, 0)),
        ],
        out_shape=[jax.ShapeDtypeStruct((n_nodes, hid2), jnp.float32)] * 2,
    )(p1, p1, r1, b1.reshape(1, hid1), w2)

    zeros2 = jnp.zeros((a_rows, hid2), jnp.float32)
    p2 = _make_edge_aggregate(n_nodes, hid2, ec)(y2, src_p, dst_p, zeros2)

    out = pl.pallas_call(
        functools.partial(_final_kern, n_graphs=n_graphs),
        grid=(1,),
        in_specs=[
            pl.BlockSpec((1, n_nodes, hid2), lambda i: (0, 0, 0)),
            pl.BlockSpec((1, n_nodes, hid2), lambda i: (1, 0, 0)),
            pl.BlockSpec((n_nodes, hid2), lambda i: (0, 0)),
            pl.BlockSpec((1, hid2), lambda i: (0, 0)),
            pl.BlockSpec((1, n_nodes), lambda i: (0, 0)),
            pl.BlockSpec((hid2, n_classes), lambda i: (0, 0)),
            pl.BlockSpec((1, n_classes), lambda i: (0, 0)),
        ],
        out_specs=pl.BlockSpec((n_graphs, n_classes), lambda i: (0, 0)),
        out_shape=jax.ShapeDtypeStruct((n_graphs, n_classes), jnp.float32),
    )(p2, p2, r2, b2.reshape(1, hid2), batch.astype(jnp.int32).reshape(1, n_nodes),
      W_fc, b_fc.reshape(1, n_classes))
    return out
